# baseline (speedup 1.0000x reference)
import functools

import jax
import jax.numpy as jnp
from jax import lax
from jax.experimental import pallas as pl
from jax.experimental.pallas import tpu as pltpu
from jax.experimental.pallas import tpu_sc as plsc

_VOCAB = 100000
_EMB = 64
_CTX10 = 10
_B = 4096
_HID = 128
_VT = 512
_NVT = (_VOCAB + _VT - 1) // _VT

_NROWS = _B * _CTX10
_CHUNK = 128
_NW = 32
_ROWS_PER_W = _NROWS // _NW
_NCH = _ROWS_PER_W // _CHUNK


def _sc_gather_body(idx_hbm, table_hbm, out_hbm, idx_v, rows_v, sem):
    nc = 2
    wid = lax.axis_index("s") * nc + lax.axis_index("c")
    base = wid * _ROWS_PER_W
    pltpu.sync_copy(idx_hbm.at[wid], idx_v)
    for i in range(_NCH):
        pltpu.async_copy(
            table_hbm.at[idx_v.at[i]],
            rows_v.at[pl.ds(i * _CHUNK, _CHUNK)],
            sem,
        ).wait()
    pltpu.sync_copy(rows_v, out_hbm.at[pl.ds(base, _ROWS_PER_W)])


def _sc_gather(idx, table):
    mesh = plsc.VectorSubcoreMesh(core_axis_name="c", subcore_axis_name="s")
    k = functools.partial(
        pl.kernel,
        mesh=mesh,
        out_type=jax.ShapeDtypeStruct((_NROWS, _EMB), jnp.float32),
        scratch_types=[
            pltpu.VMEM((_NCH, _CHUNK), jnp.int32),
            pltpu.VMEM((_ROWS_PER_W, _EMB), jnp.float32),
            pltpu.SemaphoreType.DMA,
        ],
        compiler_params=pltpu.CompilerParams(use_tc_tiling_on_sc=False),
    )(_sc_gather_body)
    return k(idx, table)


def _tc_body(emb_ref, w1_ref, b1_ref, w2_ref, b2_ref, out_ref,
             h_ref, m_ref, s_ref):
    p = pl.program_id(0)
    j = pl.program_id(1)

    @pl.when(jnp.logical_and(p == 0, j == 0))
    def _init():
        h = jnp.dot(emb_ref[...].astype(jnp.bfloat16), w1_ref[...],
                    preferred_element_type=jnp.float32) + b1_ref[...]
        h_ref[...] = jnp.maximum(h, 0.0).astype(jnp.bfloat16)
        m_ref[...] = jnp.full((_B, 1), -1e30, jnp.float32)
        s_ref[...] = jnp.zeros((_B, 1), jnp.float32)

    logits = jnp.dot(h_ref[...], w2_ref[...],
                     preferred_element_type=jnp.float32) + b2_ref[...]

    @pl.when(p == 0)
    def _stats():
        col = j * _VT + lax.broadcasted_iota(jnp.int32, (1, _VT), 1)
        lm = jnp.where(col < _VOCAB, logits, -1e30)
        m_old = m_ref[...]
        m_new = jnp.maximum(m_old, jnp.max(lm, axis=1, keepdims=True))
        s_ref[...] = (s_ref[...] * jnp.exp(m_old - m_new)
                      + jnp.sum(jnp.exp(lm - m_new), axis=1, keepdims=True))
        m_ref[...] = m_new

    @pl.when(p == 1)
    def _write():
        out_ref[...] = logits - (m_ref[...] + jnp.log(s_ref[...]))


def _tc_mlp_logsoftmax(embeds, W1, b1, W2bf, b2):
    return pl.pallas_call(
        _tc_body,
        grid=(2, _NVT),
        in_specs=[
            pl.BlockSpec((_B, _CTX10 * _EMB), lambda p, j: (0, 0)),
            pl.BlockSpec((_CTX10 * _EMB, _HID), lambda p, j: (0, 0)),
            pl.BlockSpec((1, _HID), lambda p, j: (0, 0)),
            pl.BlockSpec((_HID, _VT), lambda p, j: (0, j)),
            pl.BlockSpec((1, _VT), lambda p, j: (0, j)),
        ],
        out_specs=pl.BlockSpec((_B, _VT), lambda p, j: (0, j * p)),
        out_shape=jax.ShapeDtypeStruct((_B, _VOCAB), jnp.float32),
        scratch_shapes=[
            pltpu.VMEM((_B, _HID), jnp.bfloat16),
            pltpu.VMEM((_B, 1), jnp.float32),
            pltpu.VMEM((_B, 1), jnp.float32),
        ],
        compiler_params=pltpu.CompilerParams(
            dimension_semantics=("arbitrary", "arbitrary"),
        ),
    )(embeds, W1, b1, W2bf, b2)


def kernel(inputs, emb, W1, b1, W2, b2):
    idx = inputs.reshape(_NW, _NCH, _CHUNK)
    gathered = _sc_gather(idx, emb)
    embeds = gathered.reshape(_B, _CTX10 * _EMB)
    return _tc_mlp_logsoftmax(
        embeds,
        W1.astype(jnp.bfloat16),
        b1.reshape(1, _HID),
        W2.astype(jnp.bfloat16),
        b2.reshape(1, _VOCAB),
    )


# speedup vs baseline: 1.1219x; 1.1219x over previous
import functools

import jax
import jax.numpy as jnp
from jax import lax
from jax.experimental import pallas as pl
from jax.experimental.pallas import tpu as pltpu
from jax.experimental.pallas import tpu_sc as plsc

_VOCAB = 100000
_EMB = 64
_CTX10 = 10
_B = 4096
_HID = 128
_VT = 512
_NVT = (_VOCAB + _VT - 1) // _VT
_VPAD = _NVT * _VT - _VOCAB

_LOG2E = 1.4426950408889634
_LN2 = 0.6931471805599453

_NROWS = _B * _CTX10
_CHUNK = 128
_NW = 32
_ROWS_PER_W = _NROWS // _NW
_NCH = _ROWS_PER_W // _CHUNK


def _sc_gather_body(idx_hbm, table_hbm, out_hbm, idx_v, rows_v, sem):
    nc = 2
    wid = lax.axis_index("s") * nc + lax.axis_index("c")
    base = wid * _ROWS_PER_W
    pltpu.sync_copy(idx_hbm.at[wid], idx_v)
    for i in range(_NCH):
        pltpu.async_copy(
            table_hbm.at[idx_v.at[i]],
            rows_v.at[pl.ds(i * _CHUNK, _CHUNK)],
            sem,
        ).wait()
    pltpu.sync_copy(rows_v, out_hbm.at[pl.ds(base, _ROWS_PER_W)])


def _sc_gather(idx, table):
    mesh = plsc.VectorSubcoreMesh(core_axis_name="c", subcore_axis_name="s")
    k = functools.partial(
        pl.kernel,
        mesh=mesh,
        out_type=jax.ShapeDtypeStruct((_NROWS, _EMB), jnp.float32),
        scratch_types=[
            pltpu.VMEM((_NCH, _CHUNK), jnp.int32),
            pltpu.VMEM((_ROWS_PER_W, _EMB), jnp.float32),
            pltpu.SemaphoreType.DMA,
        ],
        compiler_params=pltpu.CompilerParams(use_tc_tiling_on_sc=False),
    )(_sc_gather_body)
    return k(idx, table)


def _h_body(emb_ref, w1_ref, b1_ref, h_ref):
    h = jnp.dot(emb_ref[...].astype(jnp.bfloat16), w1_ref[...],
                preferred_element_type=jnp.float32) + b1_ref[...]
    h_ref[...] = jnp.maximum(h, 0.0).astype(jnp.bfloat16)


def _h_kernel(embeds, W1bf, b1):
    return pl.pallas_call(
        _h_body,
        out_shape=jax.ShapeDtypeStruct((_B, _HID), jnp.bfloat16),
    )(embeds, W1bf, b1)


def _groups(x, op):
    return op(op(x[:, 0:128], x[:, 128:256]),
              op(x[:, 256:384], x[:, 384:512]))


def _passA_body(h_ref, w2_ref, b2_ref, m128_ref):
    j = pl.program_id(0)
    x = jnp.dot(h_ref[...], w2_ref[...],
                preferred_element_type=jnp.float32) + b2_ref[...]
    t = _groups(x, jnp.maximum)
    m128_ref[...] = jnp.where(j == 0, t, jnp.maximum(m128_ref[...], t))


def _passB_body(h_ref, w2_ref, b2_ref, m128_ref, s128_ref, m1_ref):
    j = pl.program_id(0)

    @pl.when(j == 0)
    def _():
        m1_ref[...] = jnp.max(m128_ref[...], axis=1, keepdims=True)

    x = jnp.dot(h_ref[...], w2_ref[...],
                preferred_element_type=jnp.float32) + b2_ref[...]
    e = jnp.exp2(x - m1_ref[...])
    t = _groups(e, jnp.add)
    s128_ref[...] = jnp.where(j == 0, t, s128_ref[...] + t)


def _passC_body(h_ref, w2_ref, b2_ref, m128_ref, s128_ref, out_ref, lse_ref):
    j = pl.program_id(0)

    @pl.when(j == 0)
    def _():
        m1 = jnp.max(m128_ref[...], axis=1, keepdims=True)
        s1 = jnp.sum(s128_ref[...], axis=1, keepdims=True)
        lse_ref[...] = m1 + jnp.log2(s1)

    x = jnp.dot(h_ref[...], w2_ref[...],
                preferred_element_type=jnp.float32) + b2_ref[...]
    out_ref[...] = (x - lse_ref[...]) * _LN2


_H_SPEC = pl.BlockSpec((_B, _HID), lambda j: (0, 0))
_W2_SPEC = pl.BlockSpec((_HID, _VT), lambda j: (0, j))
_B2_SPEC = pl.BlockSpec((1, _VT), lambda j: (0, j))
_L128_SPEC = pl.BlockSpec((_B, 128), lambda j: (0, 0))
_SEQ = pltpu.CompilerParams(dimension_semantics=("arbitrary",))


def _passA(h, W2p, b2p):
    return pl.pallas_call(
        _passA_body,
        grid=(_NVT,),
        in_specs=[_H_SPEC, _W2_SPEC, _B2_SPEC],
        out_specs=_L128_SPEC,
        out_shape=jax.ShapeDtypeStruct((_B, 128), jnp.float32),
        compiler_params=_SEQ,
    )(h, W2p, b2p)


def _passB(h, W2p, b2p, m128):
    return pl.pallas_call(
        _passB_body,
        grid=(_NVT,),
        in_specs=[_H_SPEC, _W2_SPEC, _B2_SPEC, _L128_SPEC],
        out_specs=_L128_SPEC,
        out_shape=jax.ShapeDtypeStruct((_B, 128), jnp.float32),
        scratch_shapes=[pltpu.VMEM((_B, 1), jnp.float32)],
        compiler_params=_SEQ,
    )(h, W2p, b2p, m128)


def _passC(h, W2p, b2p, m128, s128):
    return pl.pallas_call(
        _passC_body,
        grid=(_NVT,),
        in_specs=[_H_SPEC, _W2_SPEC, _B2_SPEC, _L128_SPEC, _L128_SPEC],
        out_specs=pl.BlockSpec((_B, _VT), lambda j: (0, j)),
        out_shape=jax.ShapeDtypeStruct((_B, _VOCAB), jnp.float32),
        scratch_shapes=[pltpu.VMEM((_B, 1), jnp.float32)],
        compiler_params=_SEQ,
    )(h, W2p, b2p, m128, s128)


def kernel(inputs, emb, W1, b1, W2, b2):
    idx = inputs.reshape(_NW, _NCH, _CHUNK)
    gathered = _sc_gather(idx, emb)
    embeds = gathered.reshape(_B, _CTX10 * _EMB)
    h = _h_kernel(embeds, W1.astype(jnp.bfloat16), b1.reshape(1, _HID))
    W2p = jnp.pad((W2 * _LOG2E).astype(jnp.bfloat16), ((0, 0), (0, _VPAD)))
    b2p = jnp.pad((b2 * _LOG2E).reshape(1, _VOCAB), ((0, 0), (0, _VPAD)),
                  constant_values=-1e30)
    m128 = _passA(h, W2p, b2p)
    s128 = _passB(h, W2p, b2p, m128)
    return _passC(h, W2p, b2p, m128, s128)


# speedup vs baseline: 2.2225x; 1.9811x over previous
import functools

import jax
import jax.numpy as jnp
from jax import lax
from jax.experimental import pallas as pl
from jax.experimental.pallas import tpu as pltpu
from jax.experimental.pallas import tpu_sc as plsc

_VOCAB = 100000
_EMB = 64
_CTX10 = 10
_B = 4096
_HID = 128
_VT = 512
_NVT = (_VOCAB + _VT - 1) // _VT
_VPAD = _NVT * _VT - _VOCAB

_LOG2E = 1.4426950408889634
_LN2 = 0.6931471805599453

_NROWS = _B * _CTX10
_CHUNK = 128
_NW = 32
_ROWS_PER_W = _NROWS // _NW
_NCH = _ROWS_PER_W // _CHUNK


def _sc_gather_body(idx_hbm, table_hbm, out_hbm, idx_v, rows_v, sem):
    nc = 2
    wid = lax.axis_index("s") * nc + lax.axis_index("c")
    base = wid * _ROWS_PER_W
    pltpu.sync_copy(idx_hbm.at[wid], idx_v)
    for i in range(_NCH):
        pltpu.async_copy(
            table_hbm.at[idx_v.at[i]],
            rows_v.at[pl.ds(i * _CHUNK, _CHUNK)],
            sem,
        ).wait()
    pltpu.sync_copy(rows_v, out_hbm.at[pl.ds(base, _ROWS_PER_W)])


def _sc_gather(idx, table):
    mesh = plsc.VectorSubcoreMesh(core_axis_name="c", subcore_axis_name="s")
    k = functools.partial(
        pl.kernel,
        mesh=mesh,
        out_type=jax.ShapeDtypeStruct((_NROWS, _EMB), jnp.float32),
        scratch_types=[
            pltpu.VMEM((_NCH, _CHUNK), jnp.int32),
            pltpu.VMEM((_ROWS_PER_W, _EMB), jnp.float32),
            pltpu.SemaphoreType.DMA,
        ],
        compiler_params=pltpu.CompilerParams(use_tc_tiling_on_sc=False),
    )(_sc_gather_body)
    return k(idx, table)


def _h_body(emb_ref, w1_ref, b1_ref, h_ref):
    h = jnp.dot(emb_ref[...].astype(jnp.bfloat16), w1_ref[...],
                preferred_element_type=jnp.float32) + b1_ref[...]
    h_ref[...] = jnp.maximum(h, 0.0).astype(jnp.bfloat16)


def _h_kernel(embeds, W1bf, b1):
    return pl.pallas_call(
        _h_body,
        out_shape=jax.ShapeDtypeStruct((_B, _HID), jnp.bfloat16),
    )(embeds, W1bf, b1)


def _chunk_reduce(x, op):
    parts = [x[k * 8:(k + 1) * 8] for k in range(_VT // 8)]
    while len(parts) > 1:
        parts = [op(parts[i], parts[i + 1]) for i in range(0, len(parts), 2)]
    return parts[0]


def _passAB_body(ht_ref, w2t_ref, b2t_ref, m1_ref, s8_ref):
    j = pl.program_id(0)
    x = jnp.dot(w2t_ref[...], ht_ref[...],
                preferred_element_type=jnp.float32) + b2t_ref[...]
    t1 = jnp.max(_chunk_reduce(x, jnp.maximum), axis=0, keepdims=True)
    m_old = m1_ref[...]
    m_new = jnp.where(j == 0, t1, jnp.maximum(m_old, t1))
    e8 = _chunk_reduce(jnp.exp2(x - m_new), jnp.add)
    s8_ref[...] = jnp.where(j == 0, e8,
                            s8_ref[...] * jnp.exp2(m_old - m_new) + e8)
    m1_ref[...] = m_new


def _passC_body(ht_ref, w2t_ref, b2t_ref, m1_ref, s8_ref, out_ref, lse_ref):
    j = pl.program_id(0)

    @pl.when(j == 0)
    def _():
        s1 = jnp.sum(s8_ref[...], axis=0, keepdims=True)
        lse_ref[...] = m1_ref[...] + jnp.log2(s1)

    x = jnp.dot(w2t_ref[...], ht_ref[...],
                preferred_element_type=jnp.float32) + b2t_ref[...]
    out_ref[...] = (x - lse_ref[...]) * _LN2


_HT_SPEC = pl.BlockSpec((_HID, _B), lambda j: (0, 0))
_W2T_SPEC = pl.BlockSpec((_VT, _HID), lambda j: (j, 0))
_B2T_SPEC = pl.BlockSpec((_VT, 1), lambda j: (j, 0))
_M1_SPEC = pl.BlockSpec((1, _B), lambda j: (0, 0))
_S8_SPEC = pl.BlockSpec((8, _B), lambda j: (0, 0))
_SEQ = pltpu.CompilerParams(dimension_semantics=("arbitrary",))


def _passAB(ht, W2tp, b2tp):
    return pl.pallas_call(
        _passAB_body,
        grid=(_NVT,),
        in_specs=[_HT_SPEC, _W2T_SPEC, _B2T_SPEC],
        out_specs=[_M1_SPEC, _S8_SPEC],
        out_shape=[jax.ShapeDtypeStruct((1, _B), jnp.float32),
                   jax.ShapeDtypeStruct((8, _B), jnp.float32)],
        compiler_params=_SEQ,
    )(ht, W2tp, b2tp)


def _passC(ht, W2tp, b2tp, m1, s8):
    return pl.pallas_call(
        _passC_body,
        grid=(_NVT,),
        in_specs=[_HT_SPEC, _W2T_SPEC, _B2T_SPEC, _M1_SPEC, _S8_SPEC],
        out_specs=pl.BlockSpec((_VT, _B), lambda j: (j, 0)),
        out_shape=jax.ShapeDtypeStruct((_VOCAB, _B), jnp.float32),
        scratch_shapes=[pltpu.VMEM((1, _B), jnp.float32)],
        compiler_params=_SEQ,
    )(ht, W2tp, b2tp, m1, s8)


def kernel(inputs, emb, W1, b1, W2, b2):
    idx = inputs.reshape(_NW, _NCH, _CHUNK)
    gathered = _sc_gather(idx, emb)
    embeds = gathered.reshape(_B, _CTX10 * _EMB)
    h = _h_kernel(embeds, W1.astype(jnp.bfloat16), b1.reshape(1, _HID))
    ht = h.T
    W2tp = jnp.pad((W2.T * _LOG2E).astype(jnp.bfloat16), ((0, _VPAD), (0, 0)))
    b2tp = jnp.pad((b2 * _LOG2E).reshape(_VOCAB, 1), ((0, _VPAD), (0, 0)),
                   constant_values=-1e30)
    m1, s8 = _passAB(ht, W2tp, b2tp)
    out_t = _passC(ht, W2tp, b2tp, m1, s8)
    return out_t.T


# speedup vs baseline: 2.8976x; 1.3038x over previous
import functools

import jax
import jax.numpy as jnp
from jax import lax
from jax.experimental import pallas as pl
from jax.experimental.pallas import tpu as pltpu
from jax.experimental.pallas import tpu_sc as plsc

_VOCAB = 100000
_EMB = 64
_CTX10 = 10
_B = 4096
_HID = 128
_VT = 512
_NVT = (_VOCAB + _VT - 1) // _VT
_VPAD = _NVT * _VT - _VOCAB

_LOG2E = 1.4426950408889634
_LN2 = 0.6931471805599453

_NROWS = _B * _CTX10
_CHUNK = 128
_NW = 32
_ROWS_PER_W = _NROWS // _NW
_NCH = _ROWS_PER_W // _CHUNK


def _sc_gather_body(idx_hbm, table_hbm, out_hbm, idx_v, rows_v, sem):
    nc = 2
    wid = lax.axis_index("s") * nc + lax.axis_index("c")
    base = wid * _ROWS_PER_W
    pltpu.sync_copy(idx_hbm.at[wid], idx_v)
    for i in range(_NCH):
        pltpu.async_copy(
            table_hbm.at[idx_v.at[i]],
            rows_v.at[pl.ds(i * _CHUNK, _CHUNK)],
            sem,
        ).wait()
    pltpu.sync_copy(rows_v, out_hbm.at[pl.ds(base, _ROWS_PER_W)])


def _sc_gather(idx, table):
    mesh = plsc.VectorSubcoreMesh(core_axis_name="c", subcore_axis_name="s")
    k = functools.partial(
        pl.kernel,
        mesh=mesh,
        out_type=jax.ShapeDtypeStruct((_NROWS, _EMB), jnp.float32),
        scratch_types=[
            pltpu.VMEM((_NCH, _CHUNK), jnp.int32),
            pltpu.VMEM((_ROWS_PER_W, _EMB), jnp.float32),
            pltpu.SemaphoreType.DMA,
        ],
        compiler_params=pltpu.CompilerParams(use_tc_tiling_on_sc=False),
    )(_sc_gather_body)
    return k(idx, table)


def _h_body(emb_ref, w1_ref, b1_ref, h_ref):
    h = jnp.dot(emb_ref[...].astype(jnp.bfloat16), w1_ref[...],
                preferred_element_type=jnp.float32) + b1_ref[...]
    h_ref[...] = jnp.maximum(h, 0.0).astype(jnp.bfloat16)


def _h_kernel(embeds, W1bf, b1):
    return pl.pallas_call(
        _h_body,
        out_shape=jax.ShapeDtypeStruct((_B, _HID), jnp.bfloat16),
    )(embeds, W1bf, b1)


def _chunk_reduce(x, op):
    parts = [x[k * 8:(k + 1) * 8] for k in range(_VT // 8)]
    while len(parts) > 1:
        parts = [op(parts[i], parts[i + 1]) for i in range(0, len(parts), 2)]
    return parts[0]


def _passS_body(ht_ref, w2t_ref, b2t_ref, m1_ref, s8_ref):
    j = pl.program_id(0)
    x = jnp.dot(w2t_ref[...], ht_ref[...],
                preferred_element_type=jnp.float32) + b2t_ref[...]
    e8 = _chunk_reduce(jnp.exp2(x - m1_ref[...]), jnp.add)
    s8_ref[...] = jnp.where(j == 0, e8, s8_ref[...] + e8)


def _passC_body(ht_ref, w2t_ref, b2t_ref, m1_ref, s8_ref, out_ref, lse_ref):
    j = pl.program_id(0)

    @pl.when(j == 0)
    def _():
        s1 = jnp.sum(s8_ref[...], axis=0, keepdims=True)
        lse_ref[...] = m1_ref[...] + jnp.log2(jnp.maximum(s1, 1e-30))

    x = jnp.dot(w2t_ref[...], ht_ref[...],
                preferred_element_type=jnp.float32) + b2t_ref[...]
    out_ref[...] = (x - lse_ref[...]) * _LN2


_HT_SPEC = pl.BlockSpec((_HID, _B), lambda j: (0, 0))
_W2T_SPEC = pl.BlockSpec((_VT, _HID), lambda j: (j, 0))
_B2T_SPEC = pl.BlockSpec((_VT, 1), lambda j: (j, 0))
_M1_SPEC = pl.BlockSpec((1, _B), lambda j: (0, 0))
_S8_SPEC = pl.BlockSpec((8, _B), lambda j: (0, 0))
_SEQ = pltpu.CompilerParams(dimension_semantics=("arbitrary",))


def _passS(ht, W2tp, b2tp, m1):
    return pl.pallas_call(
        _passS_body,
        grid=(_NVT,),
        in_specs=[_HT_SPEC, _W2T_SPEC, _B2T_SPEC, _M1_SPEC],
        out_specs=_S8_SPEC,
        out_shape=jax.ShapeDtypeStruct((8, _B), jnp.float32),
        compiler_params=_SEQ,
    )(ht, W2tp, b2tp, m1)


def _passC(ht, W2tp, b2tp, m1, s8):
    return pl.pallas_call(
        _passC_body,
        grid=(_NVT,),
        in_specs=[_HT_SPEC, _W2T_SPEC, _B2T_SPEC, _M1_SPEC, _S8_SPEC],
        out_specs=pl.BlockSpec((_VT, _B), lambda j: (j, 0)),
        out_shape=jax.ShapeDtypeStruct((_VOCAB, _B), jnp.float32),
        scratch_shapes=[pltpu.VMEM((1, _B), jnp.float32)],
        compiler_params=_SEQ,
    )(ht, W2tp, b2tp, m1, s8)


def kernel(inputs, emb, W1, b1, W2, b2):
    idx = inputs.reshape(_NW, _NCH, _CHUNK)
    gathered = _sc_gather(idx, emb)
    embeds = gathered.reshape(_B, _CTX10 * _EMB)
    h = _h_kernel(embeds, W1.astype(jnp.bfloat16), b1.reshape(1, _HID))
    ht = h.T
    W2tp = jnp.pad((W2.T * _LOG2E).astype(jnp.bfloat16), ((0, _VPAD), (0, 0)))
    b2tp = jnp.pad((b2 * _LOG2E).reshape(_VOCAB, 1), ((0, _VPAD), (0, 0)),
                   constant_values=-1e30)
    g = jnp.sqrt(jnp.max(jnp.sum((W2 * _LOG2E) ** 2, axis=0)))
    hn = jnp.sqrt(jnp.sum(ht.astype(jnp.float32) ** 2, axis=0, keepdims=True))
    m1 = g * hn + jnp.max(b2) * _LOG2E
    s8 = _passS(ht, W2tp, b2tp, m1)
    out_t = _passC(ht, W2tp, b2tp, m1, s8)
    return out_t.T


# speedup vs baseline: 2.9860x; 1.0305x over previous
import functools

import jax
import jax.numpy as jnp
from jax import lax
from jax.experimental import pallas as pl
from jax.experimental.pallas import tpu as pltpu
from jax.experimental.pallas import tpu_sc as plsc

_VOCAB = 100000
_EMB = 64
_CTX10 = 10
_B = 4096
_HID = 128
_VT = 1024
_NVT = (_VOCAB + _VT - 1) // _VT
_VPAD = _NVT * _VT - _VOCAB

_LOG2E = 1.4426950408889634
_LN2 = 0.6931471805599453

_NROWS = _B * _CTX10
_CHUNK = 128
_NW = 32
_ROWS_PER_W = _NROWS // _NW
_NCH = _ROWS_PER_W // _CHUNK


def _sc_gather_body(idx_hbm, table_hbm, out_hbm, idx_v, rows_v, sem):
    nc = 2
    wid = lax.axis_index("s") * nc + lax.axis_index("c")
    base = wid * _ROWS_PER_W
    pltpu.sync_copy(idx_hbm.at[wid], idx_v)
    for i in range(_NCH):
        pltpu.async_copy(
            table_hbm.at[idx_v.at[i]],
            rows_v.at[pl.ds(i * _CHUNK, _CHUNK)],
            sem,
        ).wait()
    pltpu.sync_copy(rows_v, out_hbm.at[pl.ds(base, _ROWS_PER_W)])


def _sc_gather(idx, table):
    mesh = plsc.VectorSubcoreMesh(core_axis_name="c", subcore_axis_name="s")
    k = functools.partial(
        pl.kernel,
        mesh=mesh,
        out_type=jax.ShapeDtypeStruct((_NROWS, _EMB), jnp.float32),
        scratch_types=[
            pltpu.VMEM((_NCH, _CHUNK), jnp.int32),
            pltpu.VMEM((_ROWS_PER_W, _EMB), jnp.float32),
            pltpu.SemaphoreType.DMA,
        ],
        compiler_params=pltpu.CompilerParams(use_tc_tiling_on_sc=False),
    )(_sc_gather_body)
    return k(idx, table)


def _h_body(emb_ref, w1_ref, b1_ref, h_ref):
    h = jnp.dot(emb_ref[...].astype(jnp.bfloat16), w1_ref[...],
                preferred_element_type=jnp.float32) + b1_ref[...]
    h_ref[...] = jnp.maximum(h, 0.0).astype(jnp.bfloat16)


def _h_kernel(embeds, W1bf, b1):
    return pl.pallas_call(
        _h_body,
        out_shape=jax.ShapeDtypeStruct((_B, _HID), jnp.bfloat16),
    )(embeds, W1bf, b1)


def _chunk_reduce(x, op):
    parts = [x[k * 8:(k + 1) * 8] for k in range(_VT // 8)]
    while len(parts) > 1:
        parts = [op(parts[i], parts[i + 1]) for i in range(0, len(parts), 2)]
    return parts[0]


def _passS_body(ht_ref, w2t_ref, b2t_ref, m1_ref, s8_ref):
    j = pl.program_id(0)
    x = jnp.dot(w2t_ref[...], ht_ref[...],
                preferred_element_type=jnp.float32) + b2t_ref[...]
    e8 = _chunk_reduce(jnp.exp2(x - m1_ref[...]), jnp.add)
    s8_ref[...] = jnp.where(j == 0, e8, s8_ref[...] + e8)


def _passC_body(ht_ref, w2t_ref, b2t_ref, m1_ref, s8_ref, out_ref, lse_ref):
    j = pl.program_id(0)

    @pl.when(j == 0)
    def _():
        s1 = jnp.sum(s8_ref[...], axis=0, keepdims=True)
        lse_ref[...] = m1_ref[...] + jnp.log2(jnp.maximum(s1, 1e-30))

    x = jnp.dot(w2t_ref[...], ht_ref[...],
                preferred_element_type=jnp.float32) + b2t_ref[...]
    out_ref[...] = (x - lse_ref[...]) * _LN2


_HT_SPEC = pl.BlockSpec((_HID, _B), lambda j: (0, 0))
_W2T_SPEC = pl.BlockSpec((_VT, _HID), lambda j: (j, 0))
_B2T_SPEC = pl.BlockSpec((_VT, 1), lambda j: (j, 0))
_M1_SPEC = pl.BlockSpec((1, _B), lambda j: (0, 0))
_S8_SPEC = pl.BlockSpec((8, _B), lambda j: (0, 0))
_SEQ = pltpu.CompilerParams(dimension_semantics=("arbitrary",))


def _passS(ht, W2tp, b2tp, m1):
    return pl.pallas_call(
        _passS_body,
        grid=(_NVT,),
        in_specs=[_HT_SPEC, _W2T_SPEC, _B2T_SPEC, _M1_SPEC],
        out_specs=_S8_SPEC,
        out_shape=jax.ShapeDtypeStruct((8, _B), jnp.float32),
        compiler_params=_SEQ,
    )(ht, W2tp, b2tp, m1)


def _passC(ht, W2tp, b2tp, m1, s8):
    return pl.pallas_call(
        _passC_body,
        grid=(_NVT,),
        in_specs=[_HT_SPEC, _W2T_SPEC, _B2T_SPEC, _M1_SPEC, _S8_SPEC],
        out_specs=pl.BlockSpec((_VT, _B), lambda j: (j, 0)),
        out_shape=jax.ShapeDtypeStruct((_VOCAB, _B), jnp.float32),
        scratch_shapes=[pltpu.VMEM((1, _B), jnp.float32)],
        compiler_params=_SEQ,
    )(ht, W2tp, b2tp, m1, s8)


def kernel(inputs, emb, W1, b1, W2, b2):
    idx = inputs.reshape(_NW, _NCH, _CHUNK)
    gathered = _sc_gather(idx, emb)
    embeds = gathered.reshape(_B, _CTX10 * _EMB)
    h = _h_kernel(embeds, W1.astype(jnp.bfloat16), b1.reshape(1, _HID))
    ht = h.T
    W2tp = jnp.pad((W2.T * _LOG2E).astype(jnp.bfloat16), ((0, _VPAD), (0, 0)))
    b2tp = jnp.pad((b2 * _LOG2E).reshape(_VOCAB, 1), ((0, _VPAD), (0, 0)),
                   constant_values=-1e30)
    g = jnp.sqrt(jnp.max(jnp.sum((W2 * _LOG2E) ** 2, axis=0)))
    hn = jnp.sqrt(jnp.sum(ht.astype(jnp.float32) ** 2, axis=0, keepdims=True))
    m1 = g * hn + jnp.max(b2) * _LOG2E
    s8 = _passS(ht, W2tp, b2tp, m1)
    out_t = _passC(ht, W2tp, b2tp, m1, s8)
    return out_t.T
